# baseline (speedup 1.0000x reference)
import functools

import jax
import jax.numpy as jnp
from jax import lax
from jax.experimental import pallas as pl
from jax.experimental.pallas import tpu as pltpu
from jax.experimental.pallas import tpu_sc as plsc

B = 16384
A = 200
NC = 2
NS = 16
L = 16
NW = NC * NS
RPW = B // NW
CHUNK = 128
NCH = RPW // CHUNK
CGROUPS = CHUNK // L
NACC = 8


def _take16(table_vec, idx):
    return lax.gather(
        table_vec,
        idx[:, None],
        dimension_numbers=lax.GatherDimensionNumbers(
            offset_dims=(),
            collapsed_slice_dims=(0,),
            start_index_map=(0,),
        ),
        slice_sizes=(1,),
        mode=lax.GatherScatterMode.PROMISE_IN_BOUNDS,
    )


def _sae_body(energies_hbm, species_hbm, table_hbm, out_hbm,
              buf0, buf1, energies_v, out_v, table_v, sem0, sem1):
    wid = lax.axis_index("s") * NC + lax.axis_index("c")
    base = wid * RPW

    bufs = (buf0, buf1)
    sems = (sem0, sem1)

    def chunk_copy(c, nbuf):
        return pltpu.make_async_copy(
            species_hbm.at[pl.ds(base + c * CHUNK, CHUNK)],
            bufs[nbuf], sems[nbuf])

    def start_chunk(c, nbuf):
        chunk_copy(c, nbuf).start()

    def wait_chunk(c, nbuf):
        chunk_copy(c, nbuf).wait()

    start_chunk(0, 0)
    pltpu.sync_copy(table_hbm, table_v)
    pltpu.sync_copy(energies_hbm.at[pl.ds(base, RPW)], energies_v)

    table_vec = table_v[...]
    iota = lax.iota(jnp.int32, L)
    zeros_f = jnp.zeros((L,), jnp.float32)

    def do_chunk(c, nbuf):
        wait_chunk(c, nbuf)
        species_v = bufs[nbuf]

        def group_fn(g, _):
            rows = iota + g * L

            @plsc.parallel_loop(0, A - L, carry=(iota, (zeros_f,) * NACC),
                                unroll=8)
            def loop1(_, carry):
                col, accs = carry
                s = plsc.load_gather(species_v, [rows, col])
                v = _take16(table_vec, s)
                return col + 1, accs[1:] + (accs[0] + v,)

            col1, accs1 = loop1

            @plsc.parallel_loop(0, L, carry=(col1, accs1), unroll=8)
            def loop2(_, carry):
                col, accs = carry
                cw = jnp.where(col >= A, col - A, col)
                s = plsc.load_gather(species_v, [rows, cw])
                v = _take16(table_vec, s)
                return col + 1, accs[1:] + (accs[0] + v,)

            _, accs = loop2
            acc = ((accs[0] + accs[1]) + (accs[2] + accs[3])) + (
                (accs[4] + accs[5]) + (accs[6] + accs[7]))
            off = c * CHUNK + g * L
            e = energies_v[pl.ds(off, L)]
            out_v[pl.ds(off, L)] = e - acc
            return 0

        lax.fori_loop(0, CGROUPS, group_fn, 0)

    for c in range(NCH):
        if c + 1 < NCH:
            start_chunk(c + 1, (c + 1) % 2)
        do_chunk(c, c % 2)

    pltpu.sync_copy(out_v, out_hbm.at[pl.ds(base, RPW)])


@jax.jit
def _sae_kernel(energies, species, table16):
    mesh = plsc.VectorSubcoreMesh(
        core_axis_name="c", subcore_axis_name="s",
        num_cores=NC, num_subcores=NS,
    )
    f = functools.partial(
        pl.kernel,
        mesh=mesh,
        compiler_params=pltpu.CompilerParams(needs_layout_passes=False),
        out_type=jax.ShapeDtypeStruct((B,), jnp.float32),
        scratch_types=[
            pltpu.VMEM((CHUNK, A), jnp.int32),
            pltpu.VMEM((CHUNK, A), jnp.int32),
            pltpu.VMEM((RPW,), jnp.float32),
            pltpu.VMEM((RPW,), jnp.float32),
            pltpu.VMEM((L,), jnp.float32),
            pltpu.SemaphoreType.DMA,
            pltpu.SemaphoreType.DMA,
        ],
    )(_sae_body)
    return f(energies, species, table16)


def kernel(energies, species, self_energies):
    table16 = jnp.zeros((L,), jnp.float32).at[: self_energies.shape[0]].set(
        self_energies.astype(jnp.float32))
    return _sae_kernel(energies, species.astype(jnp.int32), table16)


# speedup vs baseline: 1.0192x; 1.0192x over previous
import functools

import jax
import jax.numpy as jnp
from jax import lax
from jax.experimental import pallas as pl
from jax.experimental.pallas import tpu as pltpu
from jax.experimental.pallas import tpu_sc as plsc

B = 16384
A = 200
NC = 2
NS = 16
L = 16
NW = NC * NS

SC_B = 2048
RPW = SC_B // NW
CGROUPS = RPW // L
NACC = 8

TC_B = B - SC_B
TC_BLK = 2048


def _take16(table_vec, idx):
    return lax.gather(
        table_vec,
        idx[:, None],
        dimension_numbers=lax.GatherDimensionNumbers(
            offset_dims=(),
            collapsed_slice_dims=(0,),
            start_index_map=(0,),
        ),
        slice_sizes=(1,),
        mode=lax.GatherScatterMode.PROMISE_IN_BOUNDS,
    )


def _sc_body(energies_hbm, species_hbm, table_hbm, out_hbm,
             species_v, energies_v, out_v, table_v):
    wid = lax.axis_index("s") * NC + lax.axis_index("c")
    base = wid * RPW

    pltpu.sync_copy(species_hbm.at[pl.ds(base, RPW)], species_v)
    pltpu.sync_copy(table_hbm, table_v)
    pltpu.sync_copy(energies_hbm.at[pl.ds(base, RPW)], energies_v)

    table_vec = table_v[...]
    iota = lax.iota(jnp.int32, L)
    zeros_f = jnp.zeros((L,), jnp.float32)

    def group_fn(g, _):
        rows = iota + g * L

        @plsc.parallel_loop(0, A - L, carry=(iota, (zeros_f,) * NACC),
                            unroll=8)
        def loop1(_, carry):
            col, accs = carry
            s = plsc.load_gather(species_v, [rows, col])
            v = _take16(table_vec, s)
            return col + 1, accs[1:] + (accs[0] + v,)

        col1, accs1 = loop1

        @plsc.parallel_loop(0, L, carry=(col1, accs1), unroll=8)
        def loop2(_, carry):
            col, accs = carry
            cw = jnp.where(col >= A, col - A, col)
            s = plsc.load_gather(species_v, [rows, cw])
            v = _take16(table_vec, s)
            return col + 1, accs[1:] + (accs[0] + v,)

        _, accs = loop2
        acc = ((accs[0] + accs[1]) + (accs[2] + accs[3])) + (
            (accs[4] + accs[5]) + (accs[6] + accs[7]))
        off = g * L
        e = energies_v[pl.ds(off, L)]
        out_v[pl.ds(off, L)] = e - acc
        return 0

    lax.fori_loop(0, CGROUPS, group_fn, 0)
    pltpu.sync_copy(out_v, out_hbm.at[pl.ds(base, RPW)])


def _sc_part(energies, species, table16):
    mesh = plsc.VectorSubcoreMesh(
        core_axis_name="c", subcore_axis_name="s",
        num_cores=NC, num_subcores=NS,
    )
    f = functools.partial(
        pl.kernel,
        mesh=mesh,
        compiler_params=pltpu.CompilerParams(needs_layout_passes=False),
        out_type=jax.ShapeDtypeStruct((SC_B,), jnp.float32),
        scratch_types=[
            pltpu.VMEM((RPW, A), jnp.int32),
            pltpu.VMEM((RPW,), jnp.float32),
            pltpu.VMEM((RPW,), jnp.float32),
            pltpu.VMEM((L,), jnp.float32),
        ],
    )(_sc_body)
    return f(energies, species, table16)


def _tc_body(table_ref, energies_ref, species_ref, out_ref):
    t0 = table_ref[0]
    d1 = table_ref[1] - t0
    d2 = table_ref[2] - t0
    d3 = table_ref[3] - t0
    s = species_ref[...]
    val = jnp.where(s == 1, d1, 0.0)
    val = val + jnp.where(s == 2, d2, 0.0)
    val = val + jnp.where(s == 3, d3, 0.0)
    sae = jnp.sum(val, axis=-1) + jnp.float32(A) * t0
    out_ref[...] = energies_ref[...] - sae


def _tc_part(energies, species, table4):
    grid = (TC_B // TC_BLK,)
    off = SC_B // TC_BLK
    return pl.pallas_call(
        _tc_body,
        grid_spec=pltpu.PrefetchScalarGridSpec(
            num_scalar_prefetch=1,
            grid=grid,
            in_specs=[
                pl.BlockSpec((TC_BLK,), lambda i, t: (i + off,)),
                pl.BlockSpec((TC_BLK, A), lambda i, t: (i + off, 0)),
            ],
            out_specs=pl.BlockSpec((TC_BLK,), lambda i, t: (i,)),
        ),
        out_shape=jax.ShapeDtypeStruct((TC_B,), jnp.float32),
    )(table4, energies, species)


@jax.jit
def _sae_kernel(energies, species, table16, table4):
    sc_out = _sc_part(energies, species, table16)
    tc_out = _tc_part(energies, species, table4)
    return jnp.concatenate([sc_out, tc_out])


def kernel(energies, species, self_energies):
    table4 = self_energies.astype(jnp.float32)
    table16 = jnp.zeros((L,), jnp.float32).at[:4].set(table4)
    return _sae_kernel(energies, species.astype(jnp.int32), table16, table4)
